# baseline (speedup 1.0000x reference)
import functools

import jax
import jax.numpy as jnp
from jax import lax
from jax.experimental import pallas as pl
from jax.experimental.pallas import tpu as pltpu
from jax.experimental.pallas import tpu_sc as plsc

NC = 2
NS = 16
NW = NC * NS
BLK = 128
NBUF = 4
NBUF_SS = 2


def _pad_to(arr, n, fill):
    pad = n - arr.shape[0]
    if pad == 0:
        return arr
    return jnp.concatenate([arr, jnp.full((pad,), fill, arr.dtype)], axis=0)


def _sc_gather(table, idx2d):
    V, D = table.shape
    nblk = idx2d.shape[1]
    B = NW * nblk * BLK
    mesh = plsc.VectorSubcoreMesh(core_axis_name="c", subcore_axis_name="s")

    @functools.partial(
        pl.kernel,
        mesh=mesh,
        out_type=jax.ShapeDtypeStruct((B, D), table.dtype),
        scratch_types=[
            pltpu.VMEM((nblk, 1, BLK), jnp.int32),
            pltpu.VMEM((NBUF, BLK, D), table.dtype),
            pltpu.SemaphoreType.DMA((NBUF,)),
            pltpu.SemaphoreType.DMA((NBUF,)),
        ],
    )
    def k(table_hbm, idx_hbm, out_hbm, idx_v, rows_v, gsem, wsem):
        wid = lax.axis_index("s") * NC + lax.axis_index("c")
        pltpu.sync_copy(idx_hbm.at[wid], idx_v)
        base_row = wid * nblk * BLK

        for b in range(NBUF):
            pltpu.async_copy(table_hbm.at[idx_v.at[b, 0]], rows_v.at[b],
                             gsem.at[b])

        @pl.loop(0, nblk, step=NBUF)
        def _(s):
            for b in range(NBUF):
                i = s + b
                pltpu.make_async_copy(table_hbm.at[idx_v.at[b, 0]],
                                      rows_v.at[b], gsem.at[b]).wait()
                off = base_row + i * BLK
                pltpu.async_copy(rows_v.at[b],
                                 out_hbm.at[pl.ds(off, BLK), :], wsem.at[b])
            for b in range(NBUF):
                i = s + b + NBUF

                @pl.when(i < nblk)
                def _():
                    pltpu.make_async_copy(rows_v.at[b],
                                          out_hbm.at[pl.ds(0, BLK), :],
                                          wsem.at[b]).wait()
                    pltpu.async_copy(table_hbm.at[idx_v.at[i, 0]],
                                     rows_v.at[b], gsem.at[b])

        for b in range(NBUF):
            pltpu.make_async_copy(rows_v.at[b], out_hbm.at[pl.ds(0, BLK), :],
                                  wsem.at[b]).wait()

    return k(table, idx2d)


def _sc_segment_sum(vals, dst2d, n_nodes):
    E, D = vals.shape
    nblk = dst2d.shape[1]
    rows_per_sub = n_nodes // NS
    mesh = plsc.VectorSubcoreMesh(core_axis_name="c", subcore_axis_name="s")

    @functools.partial(
        pl.kernel,
        mesh=mesh,
        out_type=jax.ShapeDtypeStruct((NC, n_nodes, D), vals.dtype),
        scratch_types=[
            pltpu.VMEM((nblk, 1, BLK), jnp.int32),
            pltpu.VMEM((NBUF_SS, BLK, D), vals.dtype),
            pltpu.VMEM_SHARED((n_nodes, D), vals.dtype),
            pltpu.SemaphoreType.DMA((NBUF_SS,)),
            pltpu.SemaphoreType.DMA((NBUF_SS,)),
        ],
    )
    def k(vals_hbm, dst_hbm, out_hbm, idx_v, rows_v, agg_sh, gsem, ssem):
        cid = lax.axis_index("c")
        sid = lax.axis_index("s")
        wid = sid * NC + cid

        @pl.loop(0, BLK)
        def _(i):
            @pl.loop(0, D // 16)
            def _(j):
                rows_v[0, i, pl.ds(j * 16, 16)] = jnp.zeros((16,), vals.dtype)

        @pl.loop(0, rows_per_sub // BLK)
        def _(i):
            r0 = sid * rows_per_sub + i * BLK
            pltpu.sync_copy(rows_v.at[0], agg_sh.at[pl.ds(r0, BLK), :])

        pltpu.sync_copy(dst_hbm.at[wid], idx_v)
        plsc.subcore_barrier()

        base_row = wid * nblk * BLK

        for b in range(NBUF_SS):
            off = base_row + b * BLK
            pltpu.async_copy(vals_hbm.at[pl.ds(off, BLK), :], rows_v.at[b],
                             gsem.at[b])

        @pl.loop(0, nblk, step=NBUF_SS)
        def _(s):
            for b in range(NBUF_SS):
                pltpu.make_async_copy(vals_hbm.at[pl.ds(0, BLK), :],
                                      rows_v.at[b], gsem.at[b]).wait()
                pltpu.async_copy(rows_v.at[b], agg_sh.at[idx_v.at[s + b, 0]],
                                 ssem.at[b], add=True)
            for b in range(NBUF_SS):
                i = s + b + NBUF_SS

                @pl.when(i < nblk)
                def _():
                    pltpu.make_async_copy(rows_v.at[b],
                                          agg_sh.at[idx_v.at[s + b, 0]],
                                          ssem.at[b]).wait()
                    off = base_row + i * BLK
                    pltpu.async_copy(vals_hbm.at[pl.ds(off, BLK), :],
                                     rows_v.at[b], gsem.at[b])

        for b in range(NBUF_SS):
            pltpu.make_async_copy(rows_v.at[b], agg_sh.at[idx_v.at[b, 0]],
                                  ssem.at[b]).wait()

        plsc.subcore_barrier()

        r0 = sid * rows_per_sub
        pltpu.sync_copy(agg_sh.at[pl.ds(r0, rows_per_sub), :],
                        out_hbm.at[cid, pl.ds(r0, rows_per_sub), :])

    return k(vals, dst2d)


def _tc_scale(g, w_col):
    E, D = g.shape
    blk = 2048

    def body(g_ref, w_ref, o_ref):
        o_ref[...] = g_ref[...] * w_ref[...]

    return pl.pallas_call(
        body,
        grid=(E // blk,),
        in_specs=[pl.BlockSpec((blk, D), lambda i: (i, 0)),
                  pl.BlockSpec((blk, 1), lambda i: (i, 0))],
        out_specs=pl.BlockSpec((blk, D), lambda i: (i, 0)),
        out_shape=jax.ShapeDtypeStruct((E, D), g.dtype),
    )(g, w_col)


def _tc_linear(partials, W, b_row):
    _, N, D = partials.shape
    blk = 2048

    def body(p_ref, w_ref, b_ref, o_ref):
        s = p_ref[0] + p_ref[1]
        o_ref[...] = jnp.dot(s, w_ref[...],
                             preferred_element_type=jnp.float32) + b_ref[...]

    return pl.pallas_call(
        body,
        grid=(N // blk,),
        in_specs=[pl.BlockSpec((NC, blk, D), lambda i: (0, i, 0)),
                  pl.BlockSpec((D, D), lambda i: (0, 0)),
                  pl.BlockSpec((1, D), lambda i: (0, 0))],
        out_specs=pl.BlockSpec((blk, D), lambda i: (i, 0)),
        out_shape=jax.ShapeDtypeStruct((N, D), jnp.float32),
    )(partials, W, b_row)


def _tc_rowdot_halves(hab, c_pad):
    D = hab.shape[1]
    blk = 2048
    nhalf = c_pad // blk

    def body(a_ref, b_ref, o_ref):
        prod = a_ref[...] * b_ref[...]
        ones = jnp.ones((D, 1), jnp.float32)
        o_ref[...] = jnp.dot(prod, ones, preferred_element_type=jnp.float32)

    return pl.pallas_call(
        body,
        grid=(nhalf,),
        in_specs=[pl.BlockSpec((blk, D), lambda i: (i, 0)),
                  pl.BlockSpec((blk, D), lambda i: (i + nhalf, 0))],
        out_specs=pl.BlockSpec((blk, 1), lambda i: (i, 0)),
        out_shape=jax.ShapeDtypeStruct((c_pad, 1), jnp.float32),
    )(hab, hab)


def kernel(x, edge_index, edge_weight, edges, W, b):
    n_nodes, d = x.shape
    n_edges = edge_weight.shape[0]
    n_cand = edges.shape[1]

    unit = NW * BLK * NBUF
    e_pad = ((n_edges + unit - 1) // unit) * unit
    c_pad = ((2 * n_cand + unit - 1) // unit) * unit // 2

    src = _pad_to(edge_index[0].astype(jnp.int32), e_pad, 0)
    dst = _pad_to(edge_index[1].astype(jnp.int32), e_pad, 0)
    w_col = _pad_to(edge_weight, e_pad, 0.0).reshape(e_pad, 1)
    e0 = _pad_to(edges[0].astype(jnp.int32), c_pad, 0)
    e1 = _pad_to(edges[1].astype(jnp.int32), c_pad, 0)
    cat = jnp.concatenate([e0, e1]).reshape(NW, 2 * c_pad // (NW * BLK), 1, BLK)

    n_pad = ((n_nodes + NS * BLK - 1) // (NS * BLK)) * (NS * BLK)

    g = _sc_gather(x, src.reshape(NW, e_pad // (NW * BLK), 1, BLK))
    gw = _tc_scale(g, w_col)
    partials = _sc_segment_sum(
        gw, dst.reshape(NW, e_pad // (NW * BLK), 1, BLK), n_pad)
    h = _tc_linear(partials, W, b.reshape(1, d))
    hab = _sc_gather(h, cat)
    out = _tc_rowdot_halves(hab, c_pad)
    return out[:n_cand, 0]
